# Initial kernel scaffold; baseline (speedup 1.0000x reference)
import jax
import jax.numpy as jnp
from jax.experimental import pallas as pl


def kernel(predictions, mutex_pairs, implies_pairs):
    raise NotImplementedError("write your pallas kernel here")


# speedup vs baseline: 1.7909x; 1.7909x over previous
import functools

import jax
import jax.numpy as jnp
from jax import lax
from jax.experimental import pallas as pl
from jax.experimental.pallas import tpu as pltpu
from jax.experimental.pallas import tpu_sc as plsc

B, N = 4096, 1000
NC, NS, L = 2, 16, 16
NW = NC * NS
ROWS_PER_W = B // NW
HALF = 64
K = 4096
CH = K // L

_mesh = plsc.VectorSubcoreMesh(core_axis_name="c", subcore_axis_name="s")


@functools.partial(
    pl.kernel,
    out_type=jax.ShapeDtypeStruct((NW, L), jnp.float32),
    mesh=_mesh,
    scratch_types=[
        pltpu.VMEM((HALF * N,), jnp.float32),
        pltpu.VMEM((K,), jnp.int32),
        pltpu.VMEM((K,), jnp.int32),
        pltpu.VMEM((K,), jnp.int32),
        pltpu.VMEM((K,), jnp.int32),
        pltpu.VMEM((L,), jnp.float32),
    ],
    compiler_params=pltpu.CompilerParams(
        use_tc_tiling_on_sc=False, needs_layout_passes=False),
)
def _sc_loss(p_hbm, ma_hbm, mb_hbm, ia_hbm, ib_hbm, out_hbm,
             slab, ma, mb, ia, ib, obuf):
    wid = lax.axis_index("s") * NC + lax.axis_index("c")
    pltpu.sync_copy(ma_hbm, ma)
    pltpu.sync_copy(mb_hbm, mb)
    pltpu.sync_copy(ia_hbm, ia)
    pltpu.sync_copy(ib_hbm, ib)

    def make_chunk_body(av_ref, bv_ref, is_mutex):
        def chunk_body(c, tot):
            av = av_ref[pl.ds(c * L, L)]
            bv = bv_ref[pl.ds(c * L, L)]

            def row_body(r, acc):
                off = jnp.full((L,), r * N, jnp.int32)
                pa = plsc.load_gather(slab, [off + av])
                pb = plsc.load_gather(slab, [off + bv])
                if is_mutex:
                    return acc + pa * pb
                return acc + jnp.maximum(pa - pb, 0.0)

            return lax.fori_loop(0, HALF, row_body, tot, unroll=8)
        return chunk_body

    total = jnp.zeros((L,), jnp.float32)
    for half in range(ROWS_PER_W // HALF):
        base = (wid * ROWS_PER_W + half * HALF) * N
        pltpu.sync_copy(p_hbm.at[pl.ds(base, HALF * N)], slab)
        total = lax.fori_loop(0, CH, make_chunk_body(ma, mb, True), total)
        total = lax.fori_loop(0, CH, make_chunk_body(ia, ib, False), total)

    obuf[...] = total
    pltpu.sync_copy(obuf, out_hbm.at[wid])


def kernel(predictions, mutex_pairs, implies_pairs):
    ma = mutex_pairs[:, 0].astype(jnp.int32)
    mb = mutex_pairs[:, 1].astype(jnp.int32)
    ia = implies_pairs[:, 0].astype(jnp.int32)
    ib = implies_pairs[:, 1].astype(jnp.int32)
    partials = _sc_loss(predictions.reshape(-1), ma, mb, ia, ib)
    return jnp.sum(partials) * (1.0 / B)


# speedup vs baseline: 2.5609x; 1.4300x over previous
import functools

import jax
import jax.numpy as jnp
from jax import lax
from jax.experimental import pallas as pl
from jax.experimental.pallas import tpu as pltpu
from jax.experimental.pallas import tpu_sc as plsc

B, N = 4096, 1000
NC, NS, L = 2, 16, 16
NW = NC * NS
ROWS_PER_W = B // NW
HALF = 64
K = 4096
CH = K // L

_mesh = plsc.VectorSubcoreMesh(core_axis_name="c", subcore_axis_name="s")


@functools.partial(
    pl.kernel,
    out_type=jax.ShapeDtypeStruct((NW, L), jnp.float32),
    mesh=_mesh,
    scratch_types=[
        pltpu.VMEM((HALF * N,), jnp.float32),
        pltpu.VMEM((K,), jnp.int32),
        pltpu.VMEM((K,), jnp.int32),
        pltpu.VMEM((L,), jnp.float32),
    ],
    compiler_params=pltpu.CompilerParams(
        use_tc_tiling_on_sc=False, needs_layout_passes=False),
)
def _sc_implies(p_hbm, ia_hbm, ib_hbm, out_hbm, slab, ia, ib, obuf):
    wid = lax.axis_index("s") * NC + lax.axis_index("c")
    pltpu.sync_copy(ia_hbm, ia)
    pltpu.sync_copy(ib_hbm, ib)

    def chunk_body(c, tot):
        av = ia[pl.ds(c * L, L)]
        bv = ib[pl.ds(c * L, L)]

        def row_body(r, acc):
            off = jnp.full((L,), r * N, jnp.int32)
            pa = plsc.load_gather(slab, [off + av])
            pb = plsc.load_gather(slab, [off + bv])
            return acc + jnp.maximum(pa - pb, 0.0)

        return lax.fori_loop(0, HALF, row_body, tot, unroll=8)

    total = jnp.zeros((L,), jnp.float32)
    for half in range(ROWS_PER_W // HALF):
        base = (wid * ROWS_PER_W + half * HALF) * N
        pltpu.sync_copy(p_hbm.at[pl.ds(base, HALF * N)], slab)
        total = lax.fori_loop(0, CH, chunk_body, total)

    obuf[...] = total
    pltpu.sync_copy(obuf, out_hbm.at[wid])


BLK = 512
NBLK = B // BLK


def _tc_mutex_body(p_ref, ma_ref, mb_ref, out_ref, g_acc, m_acc):
    i = pl.program_id(0)
    pb = p_ref[...].astype(jnp.bfloat16)
    g_part = lax.dot_general(pb, pb, (((0,), (0,)), ((), ())),
                             preferred_element_type=jnp.float32)
    am = ma_ref[0, 0, :]
    bm = mb_ref[0, 0, :]
    cols = lax.broadcasted_iota(jnp.int32, (BLK, N), 1)
    a_oh = (cols == am[:, None]).astype(jnp.bfloat16)
    b_oh = (cols == bm[:, None]).astype(jnp.bfloat16)
    m_part = lax.dot_general(a_oh, b_oh, (((0,), (0,)), ((), ())),
                             preferred_element_type=jnp.float32)

    @pl.when(i == 0)
    def _():
        g_acc[...] = g_part
        m_acc[...] = m_part

    @pl.when(i > 0)
    def _():
        g_acc[...] += g_part
        m_acc[...] += m_part

    @pl.when(i == NBLK - 1)
    def _():
        out_ref[...] = jnp.sum(g_acc[...] * m_acc[...]).reshape(1, 1)


_tc_mutex = pl.pallas_call(
    _tc_mutex_body,
    grid=(NBLK,),
    in_specs=[
        pl.BlockSpec((BLK, N), lambda i: (i, 0)),
        pl.BlockSpec((1, 1, BLK), lambda i: (i, 0, 0)),
        pl.BlockSpec((1, 1, BLK), lambda i: (i, 0, 0)),
    ],
    out_specs=pl.BlockSpec((1, 1), lambda i: (0, 0)),
    out_shape=jax.ShapeDtypeStruct((1, 1), jnp.float32),
    scratch_shapes=[
        pltpu.VMEM((N, N), jnp.float32),
        pltpu.VMEM((N, N), jnp.float32),
    ],
)


def kernel(predictions, mutex_pairs, implies_pairs):
    ma = mutex_pairs[:, 0].astype(jnp.int32).reshape(NBLK, 1, BLK)
    mb = mutex_pairs[:, 1].astype(jnp.int32).reshape(NBLK, 1, BLK)
    ia = implies_pairs[:, 0].astype(jnp.int32)
    ib = implies_pairs[:, 1].astype(jnp.int32)
    partials = _sc_implies(predictions.reshape(-1), ia, ib)
    mutex_sum = _tc_mutex(predictions, ma, mb)[0, 0]
    return (jnp.sum(partials) + mutex_sum) * (1.0 / B)
